# baseline (speedup 1.0000x reference)
import functools

import jax
import jax.numpy as jnp
from jax import lax
from jax.experimental import pallas as pl
from jax.experimental.pallas import tpu as pltpu
from jax.experimental.pallas import tpu_sc as plsc

L = 16
G = 256


@functools.cache
def _make_sc_bilinear(N, HW, C, W, H):
    info = plsc.get_sparse_core_info()
    NW = info.num_cores * info.num_subcores
    npw = N // NW
    nchunks = npw // G
    NB = N // 128
    GB = G // 128
    CT = C // 8
    OROWS = CT * GB * 8
    mesh = plsc.VectorSubcoreMesh(core_axis_name="c", subcore_axis_name="s")

    @functools.partial(
        pl.kernel,
        mesh=mesh,
        compiler_params=pltpu.CompilerParams(
            use_tc_tiling_on_sc=False, needs_layout_passes=False),
        out_type=jax.ShapeDtypeStruct((CT * NB * 8, 128), jnp.float32),
        scratch_types=[
            pltpu.VMEM((2, G), jnp.float32),
            pltpu.VMEM((2, G), jnp.float32),
            pltpu.VMEM((2, G), jnp.int32),
            pltpu.VMEM((2, G), jnp.int32),
            pltpu.VMEM((2, G), jnp.int32),
            pltpu.VMEM((2, G), jnp.int32),
            pltpu.VMEM((2, G), jnp.float32),
            pltpu.VMEM((2, G), jnp.float32),
            pltpu.VMEM((2, G), jnp.float32),
            pltpu.VMEM((2, G), jnp.float32),
            pltpu.VMEM((2, G, C), jnp.float32),
            pltpu.VMEM((2, G, C), jnp.float32),
            pltpu.VMEM((2, G, C), jnp.float32),
            pltpu.VMEM((2, G, C), jnp.float32),
            pltpu.VMEM((2, OROWS, 128), jnp.float32),
            pltpu.SemaphoreType.DMA,
            pltpu.SemaphoreType.DMA,
            pltpu.SemaphoreType.DMA,
        ],
    )
    def sc_kernel(xs_hbm, ys_hbm, table_hbm, out_hbm,
                  xv, yv, i00, i01, i10, i11, w00, w01, w10, w11,
                  r00, r01, r10, r11, ov, csem, gsem, osem):
        wid = lax.axis_index("s") * info.num_cores + lax.axis_index("c")
        base0 = wid * npw

        def coords_fetch(ci):
            b = ci % 2
            base = base0 + ci * G
            pltpu.async_copy(xs_hbm.at[pl.ds(base, G)], xv.at[b], csem)
            pltpu.async_copy(ys_hbm.at[pl.ds(base, G)], yv.at[b], csem)

        def coords_wait():
            pltpu.make_async_copy(xs_hbm.at[pl.ds(0, G)], xv.at[0], csem).wait()
            pltpu.make_async_copy(ys_hbm.at[pl.ds(0, G)], yv.at[0], csem).wait()

        def prep(ci):
            b = ci % 2

            def grp(j, c2):
                s = pl.ds(j * L, L)
                x = xv[b, s]
                y = yv[b, s]
                ix = jnp.clip((x + 1.0) * (0.5 * (W - 1)), 0.0, float(W - 1))
                iy = jnp.clip((y + 1.0) * (0.5 * (H - 1)), 0.0, float(H - 1))
                x0 = ix.astype(jnp.int32)
                y0 = iy.astype(jnp.int32)
                fx = ix - x0.astype(jnp.float32)
                fy = iy - y0.astype(jnp.float32)
                x1 = jnp.minimum(x0 + 1, W - 1)
                y1 = jnp.minimum(y0 + 1, H - 1)
                b0 = y0 * W
                b1 = y1 * W
                i00[b, s] = b0 + x0
                i01[b, s] = b0 + x1
                i10[b, s] = b1 + x0
                i11[b, s] = b1 + x1
                gx = 1.0 - fx
                gy = 1.0 - fy
                w00[b, s] = gx * gy
                w01[b, s] = fx * gy
                w10[b, s] = gx * fy
                w11[b, s] = fx * fy
                return c2

            lax.fori_loop(0, G // L, grp, 0)
            pltpu.async_copy(table_hbm.at[i00.at[b]], r00.at[b], gsem)
            pltpu.async_copy(table_hbm.at[i01.at[b]], r01.at[b], gsem)
            pltpu.async_copy(table_hbm.at[i10.at[b]], r10.at[b], gsem)
            pltpu.async_copy(table_hbm.at[i11.at[b]], r11.at[b], gsem)

        def gather_wait(b):
            pltpu.make_async_copy(table_hbm.at[i00.at[b]], r00.at[b], gsem).wait()
            pltpu.make_async_copy(table_hbm.at[i01.at[b]], r01.at[b], gsem).wait()
            pltpu.make_async_copy(table_hbm.at[i10.at[b]], r10.at[b], gsem).wait()
            pltpu.make_async_copy(table_hbm.at[i11.at[b]], r11.at[b], gsem).wait()

        def blend(ci):
            b = ci % 2
            base = base0 + ci * G
            gather_wait(b)
            iota = lax.iota(jnp.int32, L)

            def grp2(j, c2):
                sw = pl.ds(j * L, L)
                w00v = w00[b, sw]
                w01v = w01[b, sw]
                w10v = w10[b, sw]
                w11v = w11[b, sw]
                rows = j * L + iota
                tcl8 = (j // 8) * 8
                lo = (j % 8) * L
                for c in range(C):
                    col = jnp.full((L,), c, jnp.int32)
                    v00 = plsc.load_gather(r00.at[b], [rows, col])
                    v01 = plsc.load_gather(r01.at[b], [rows, col])
                    v10 = plsc.load_gather(r10.at[b], [rows, col])
                    v11 = plsc.load_gather(r11.at[b], [rows, col])
                    acc = (w00v * v00 + w01v * v01 + w10v * v10 + w11v * v11)
                    orow = (c // 8) * (GB * 8) + tcl8 + (c % 8)
                    ov[b, orow, pl.ds(lo, L)] = acc
                return c2

            lax.fori_loop(0, G // L, grp2, 0)
            tcg = base // 128
            for tr in range(CT):
                pltpu.async_copy(
                    ov.at[b, pl.ds(tr * GB * 8, GB * 8), :],
                    out_hbm.at[pl.ds((tr * NB + tcg) * 8, GB * 8), :],
                    osem)

        def out_drain(b):
            for tr in range(CT):
                pltpu.make_async_copy(
                    ov.at[b, pl.ds(tr * GB * 8, GB * 8), :],
                    out_hbm.at[pl.ds(0, GB * 8), :],
                    osem).wait()

        coords_fetch(0)
        coords_fetch(1)
        coords_wait()
        prep(0)

        def body(i, carry):
            @pl.when(i + 2 < nchunks)
            def _():
                coords_fetch(i + 2)

            @pl.when(i + 1 < nchunks)
            def _():
                coords_wait()
                prep(i + 1)

            @pl.when(i >= 2)
            def _():
                out_drain(i % 2)

            blend(i)
            return carry

        lax.fori_loop(0, nchunks, body, 0)
        out_drain(nchunks % 2)
        out_drain((nchunks + 1) % 2)

    return sc_kernel


def kernel(inp, plane):
    C, H, W = plane.shape
    N = inp.shape[0]
    table = plane.transpose(1, 2, 0).reshape(H * W, C)
    xs = inp[:, 0]
    ys = inp[:, 1]
    out4 = _make_sc_bilinear(N, H * W, C, W, H)(xs, ys, table)
    NB = N // 128
    out = (out4.reshape(C // 8, NB, 8, 128)
           .transpose(0, 2, 1, 3)
           .reshape(C, N)
           .T)
    return out


# speedup vs baseline: 2.1561x; 2.1561x over previous
import functools

import jax
import jax.numpy as jnp
from jax import lax
from jax.experimental import pallas as pl
from jax.experimental.pallas import tpu as pltpu
from jax.experimental.pallas import tpu_sc as plsc

L = 16
G = 256


@functools.cache
def _make_sc_bilinear(N, HW, C, W, H):
    info = plsc.get_sparse_core_info()
    NW = info.num_cores * info.num_subcores
    npw = N // NW
    nchunks = npw // G
    NB = N // 128
    GB = G // 128
    CT = C // 8
    OROWS = CT * GB * 8
    mesh = plsc.VectorSubcoreMesh(core_axis_name="c", subcore_axis_name="s")

    @functools.partial(
        pl.kernel,
        mesh=mesh,
        compiler_params=pltpu.CompilerParams(
            use_tc_tiling_on_sc=False, needs_layout_passes=False),
        out_type=jax.ShapeDtypeStruct((CT * NB * 8, 128), jnp.float32),
        scratch_types=[
            pltpu.VMEM((2, G), jnp.float32),
            pltpu.VMEM((2, G), jnp.float32),
            pltpu.VMEM((2, G), jnp.int32),
            pltpu.VMEM((2, G), jnp.int32),
            pltpu.VMEM((2, G), jnp.int32),
            pltpu.VMEM((2, G), jnp.int32),
            pltpu.VMEM((2, G), jnp.float32),
            pltpu.VMEM((2, G), jnp.float32),
            pltpu.VMEM((2, G), jnp.float32),
            pltpu.VMEM((2, G), jnp.float32),
            pltpu.VMEM((2, G, C), jnp.float32),
            pltpu.VMEM((2, G, C), jnp.float32),
            pltpu.VMEM((2, G, C), jnp.float32),
            pltpu.VMEM((2, G, C), jnp.float32),
            pltpu.VMEM((2, OROWS, 128), jnp.float32),
            pltpu.SemaphoreType.DMA,
            pltpu.SemaphoreType.DMA,
            pltpu.SemaphoreType.DMA,
        ],
    )
    def sc_kernel(xs_hbm, ys_hbm, table_hbm, out_hbm,
                  xv, yv, i00, i01, i10, i11, w00, w01, w10, w11,
                  r00, r01, r10, r11, ov, csem, gsem, osem):
        wid = lax.axis_index("s") * info.num_cores + lax.axis_index("c")
        base0 = wid * npw

        def coords_fetch(ci):
            b = ci % 2
            base = base0 + ci * G
            pltpu.async_copy(xs_hbm.at[pl.ds(base, G)], xv.at[b], csem)
            pltpu.async_copy(ys_hbm.at[pl.ds(base, G)], yv.at[b], csem)

        def coords_wait():
            pltpu.make_async_copy(xs_hbm.at[pl.ds(0, G)], xv.at[0], csem).wait()
            pltpu.make_async_copy(ys_hbm.at[pl.ds(0, G)], yv.at[0], csem).wait()

        def prep(ci):
            b = ci % 2

            def grp(j, c2):
                s = pl.ds(j * L, L)
                x = xv[b, s]
                y = yv[b, s]
                ix = jnp.clip((x + 1.0) * (0.5 * (W - 1)), 0.0, float(W - 1))
                iy = jnp.clip((y + 1.0) * (0.5 * (H - 1)), 0.0, float(H - 1))
                x0 = ix.astype(jnp.int32)
                y0 = iy.astype(jnp.int32)
                fx = ix - x0.astype(jnp.float32)
                fy = iy - y0.astype(jnp.float32)
                x1 = jnp.minimum(x0 + 1, W - 1)
                y1 = jnp.minimum(y0 + 1, H - 1)
                b0 = y0 * W
                b1 = y1 * W
                i00[b, s] = b0 + x0
                i01[b, s] = b0 + x1
                i10[b, s] = b1 + x0
                i11[b, s] = b1 + x1
                gx = 1.0 - fx
                gy = 1.0 - fy
                w00[b, s] = gx * gy
                w01[b, s] = fx * gy
                w10[b, s] = gx * fy
                w11[b, s] = fx * fy
                return c2

            lax.fori_loop(0, G // L, grp, 0)
            pltpu.async_copy(table_hbm.at[i00.at[b]], r00.at[b], gsem)
            pltpu.async_copy(table_hbm.at[i01.at[b]], r01.at[b], gsem)
            pltpu.async_copy(table_hbm.at[i10.at[b]], r10.at[b], gsem)
            pltpu.async_copy(table_hbm.at[i11.at[b]], r11.at[b], gsem)

        def gather_wait(b):
            pltpu.make_async_copy(table_hbm.at[i00.at[b]], r00.at[b], gsem).wait()
            pltpu.make_async_copy(table_hbm.at[i01.at[b]], r01.at[b], gsem).wait()
            pltpu.make_async_copy(table_hbm.at[i10.at[b]], r10.at[b], gsem).wait()
            pltpu.make_async_copy(table_hbm.at[i11.at[b]], r11.at[b], gsem).wait()

        def blend(ci):
            b = ci % 2
            base = base0 + ci * G
            gather_wait(b)
            iota = lax.iota(jnp.int32, L)
            GB8 = GB * 8
            rowpat = (iota // 8) * GB8 + (iota % 8)

            def grp2(j, c2):
                sw = pl.ds(j * L, L)
                a00g = w00[b, sw]
                a01g = w01[b, sw]
                a10g = w10[b, sw]
                a11g = w11[b, sw]
                tcl8 = (j // 8) * 8
                lanebase = (j % 8) * L
                rowsA = rowpat + tcl8
                rowsB = rowsA + 2 * GB8
                for k in range(L):
                    p = j * L + k
                    a00 = a00g[k]
                    a01 = a01g[k]
                    a10 = a10g[k]
                    a11 = a11g[k]
                    lane = jnp.full((L,), lanebase + k, jnp.int32)
                    for ci2, rowsv in ((0, rowsA), (1, rowsB)):
                        s = pl.ds(ci2 * L, L)
                        acc = (a00 * r00[b, p, s] + a01 * r01[b, p, s]
                               + a10 * r10[b, p, s] + a11 * r11[b, p, s])
                        plsc.store_scatter(ov.at[b], [rowsv, lane], acc)
                return c2

            lax.fori_loop(0, G // L, grp2, 0)
            tcg = base // 128
            for tr in range(CT):
                pltpu.async_copy(
                    ov.at[b, pl.ds(tr * GB * 8, GB * 8), :],
                    out_hbm.at[pl.ds((tr * NB + tcg) * 8, GB * 8), :],
                    osem)

        def out_drain(b):
            for tr in range(CT):
                pltpu.make_async_copy(
                    ov.at[b, pl.ds(tr * GB * 8, GB * 8), :],
                    out_hbm.at[pl.ds(0, GB * 8), :],
                    osem).wait()

        coords_fetch(0)
        coords_fetch(1)
        coords_wait()
        prep(0)

        def body(i, carry):
            @pl.when(i + 2 < nchunks)
            def _():
                coords_fetch(i + 2)

            @pl.when(i + 1 < nchunks)
            def _():
                coords_wait()
                prep(i + 1)

            @pl.when(i >= 2)
            def _():
                out_drain(i % 2)

            blend(i)
            return carry

        lax.fori_loop(0, nchunks, body, 0)
        out_drain(nchunks % 2)
        out_drain((nchunks + 1) % 2)

    return sc_kernel


def kernel(inp, plane):
    C, H, W = plane.shape
    N = inp.shape[0]
    table = plane.transpose(1, 2, 0).reshape(H * W, C)
    xs = inp[:, 0]
    ys = inp[:, 1]
    out4 = _make_sc_bilinear(N, H * W, C, W, H)(xs, ys, table)
    NB = N // 128
    out = (out4.reshape(C // 8, NB, 8, 128)
           .transpose(0, 2, 1, 3)
           .reshape(C, N)
           .T)
    return out


# speedup vs baseline: 2.9167x; 1.3528x over previous
import functools

import jax
import jax.numpy as jnp
from jax import lax
from jax.experimental import pallas as pl
from jax.experimental.pallas import tpu as pltpu
from jax.experimental.pallas import tpu_sc as plsc

L = 16
G = 128
NBUF = 4


@functools.cache
def _make_sc_bilinear(N, HW, C, W, H):
    info = plsc.get_sparse_core_info()
    NW = info.num_cores * info.num_subcores
    npw = N // NW
    nchunks = npw // G
    NB = N // 128
    GB = G // 128
    CT = C // 8
    OROWS = CT * GB * 8
    mesh = plsc.VectorSubcoreMesh(core_axis_name="c", subcore_axis_name="s")

    @functools.partial(
        pl.kernel,
        mesh=mesh,
        compiler_params=pltpu.CompilerParams(
            use_tc_tiling_on_sc=False, needs_layout_passes=False),
        out_type=jax.ShapeDtypeStruct((CT * NB * 8, 128), jnp.float32),
        scratch_types=[
            pltpu.VMEM((NBUF, G), jnp.float32),
            pltpu.VMEM((NBUF, G), jnp.float32),
            pltpu.VMEM((NBUF, G), jnp.int32),
            pltpu.VMEM((NBUF, G), jnp.int32),
            pltpu.VMEM((NBUF, G), jnp.int32),
            pltpu.VMEM((NBUF, G), jnp.int32),
            pltpu.VMEM((NBUF, G), jnp.float32),
            pltpu.VMEM((NBUF, G), jnp.float32),
            pltpu.VMEM((NBUF, G), jnp.float32),
            pltpu.VMEM((NBUF, G), jnp.float32),
            pltpu.VMEM((NBUF, G, C), jnp.float32),
            pltpu.VMEM((NBUF, G, C), jnp.float32),
            pltpu.VMEM((NBUF, G, C), jnp.float32),
            pltpu.VMEM((NBUF, G, C), jnp.float32),
            pltpu.VMEM((2, OROWS, 128), jnp.float32),
            pltpu.SemaphoreType.DMA,
            pltpu.SemaphoreType.DMA,
            pltpu.SemaphoreType.DMA,
        ],
    )
    def sc_kernel(xs_hbm, ys_hbm, table_hbm, out_hbm,
                  xv, yv, i00, i01, i10, i11, w00, w01, w10, w11,
                  r00, r01, r10, r11, ov, csem, gsem, osem):
        wid = lax.axis_index("s") * info.num_cores + lax.axis_index("c")
        base0 = wid * npw
        tbl = table_hbm

        def coords_fetch(ci):
            b = ci % NBUF
            base = base0 + ci * G
            pltpu.async_copy(xs_hbm.at[pl.ds(base, G)], xv.at[b], csem)
            pltpu.async_copy(ys_hbm.at[pl.ds(base, G)], yv.at[b], csem)

        def coords_wait():
            pltpu.make_async_copy(xs_hbm.at[pl.ds(0, G)], xv.at[0], csem).wait()
            pltpu.make_async_copy(ys_hbm.at[pl.ds(0, G)], yv.at[0], csem).wait()

        def prep(ci):
            b = ci % NBUF

            @plsc.parallel_loop(0, G // L)
            def grp(j):
                s = pl.ds(j * L, L)
                x = xv[b, s]
                y = yv[b, s]
                ix = jnp.clip((x + 1.0) * (0.5 * (W - 1)), 0.0, float(W - 1))
                iy = jnp.clip((y + 1.0) * (0.5 * (H - 1)), 0.0, float(H - 1))
                x0 = ix.astype(jnp.int32)
                y0 = iy.astype(jnp.int32)
                fx = ix - x0.astype(jnp.float32)
                fy = iy - y0.astype(jnp.float32)
                x1 = jnp.minimum(x0 + 1, W - 1)
                y1 = jnp.minimum(y0 + 1, H - 1)
                b0 = y0 * W
                b1 = y1 * W
                i00[b, s] = b0 + x0
                i01[b, s] = b0 + x1
                i10[b, s] = b1 + x0
                i11[b, s] = b1 + x1
                gx = 1.0 - fx
                gy = 1.0 - fy
                w00[b, s] = gx * gy
                w01[b, s] = fx * gy
                w10[b, s] = gx * fy
                w11[b, s] = fx * fy

            pltpu.async_copy(tbl.at[i00.at[b]], r00.at[b], gsem)
            pltpu.async_copy(tbl.at[i01.at[b]], r01.at[b], gsem)
            pltpu.async_copy(tbl.at[i10.at[b]], r10.at[b], gsem)
            pltpu.async_copy(tbl.at[i11.at[b]], r11.at[b], gsem)

        def gather_wait(b):
            pltpu.make_async_copy(tbl.at[i00.at[b]], r00.at[b], gsem).wait()
            pltpu.make_async_copy(tbl.at[i01.at[b]], r01.at[b], gsem).wait()
            pltpu.make_async_copy(tbl.at[i10.at[b]], r10.at[b], gsem).wait()
            pltpu.make_async_copy(tbl.at[i11.at[b]], r11.at[b], gsem).wait()

        def blend(ci):
            b = ci % NBUF
            bo = ci % 2
            base = base0 + ci * G
            gather_wait(b)
            iota = lax.iota(jnp.int32, L)
            GB8 = GB * 8
            rowpat = (iota // 8) * GB8 + (iota % 8)

            @plsc.parallel_loop(0, G // L)
            def grp2(j):
                sw = pl.ds(j * L, L)
                a00g = w00[b, sw]
                a01g = w01[b, sw]
                a10g = w10[b, sw]
                a11g = w11[b, sw]
                tcl8 = (j // 8) * 8
                lanebase = (j % 8) * L
                rowsA = rowpat + tcl8
                rowsB = rowsA + 2 * GB8
                for k in range(L):
                    p = j * L + k
                    a00 = a00g[k]
                    a01 = a01g[k]
                    a10 = a10g[k]
                    a11 = a11g[k]
                    lane = jnp.full((L,), lanebase + k, jnp.int32)
                    for ci2, rowsv in ((0, rowsA), (1, rowsB)):
                        s = pl.ds(ci2 * L, L)
                        acc = (a00 * r00[b, p, s] + a01 * r01[b, p, s]
                               + a10 * r10[b, p, s] + a11 * r11[b, p, s])
                        plsc.store_scatter(ov.at[bo], [rowsv, lane], acc)

            tcg = base // 128
            for tr in range(CT):
                pltpu.async_copy(
                    ov.at[bo, pl.ds(tr * GB * 8, GB * 8), :],
                    out_hbm.at[pl.ds((tr * NB + tcg) * 8, GB * 8), :],
                    osem)

        def out_drain(b):
            for tr in range(CT):
                pltpu.make_async_copy(
                    ov.at[b, pl.ds(tr * GB * 8, GB * 8), :],
                    out_hbm.at[pl.ds(0, GB * 8), :],
                    osem).wait()

        coords_fetch(0)
        coords_fetch(1)
        coords_fetch(2)
        coords_wait()
        prep(0)
        coords_wait()
        prep(1)

        def body(i, carry):
            @pl.when(i + 3 < nchunks)
            def _():
                coords_fetch(i + 3)

            @pl.when(i + 2 < nchunks)
            def _():
                coords_wait()
                prep(i + 2)

            @pl.when(i >= 2)
            def _():
                out_drain(i % 2)

            blend(i)
            return carry

        lax.fori_loop(0, nchunks, body, 0)
        out_drain(nchunks % 2)
        out_drain((nchunks + 1) % 2)

    return sc_kernel


def kernel(inp, plane):
    C, H, W = plane.shape
    N = inp.shape[0]
    table = plane.transpose(1, 2, 0).reshape(H * W, C)
    xs = inp[:, 0]
    ys = inp[:, 1]
    out4 = _make_sc_bilinear(N, H * W, C, W, H)(xs, ys, table)
    NB = N // 128
    out = (out4.reshape(C // 8, NB, 8, 128)
           .transpose(0, 2, 1, 3)
           .reshape(C, N)
           .T)
    return out


# speedup vs baseline: 2.9195x; 1.0009x over previous
import functools

import jax
import jax.numpy as jnp
from jax import lax
from jax.experimental import pallas as pl
from jax.experimental.pallas import tpu as pltpu
from jax.experimental.pallas import tpu_sc as plsc

L = 16
G = 256


@functools.cache
def _make_sc_bilinear(N, HW, C, W, H):
    info = plsc.get_sparse_core_info()
    NW = info.num_cores * info.num_subcores
    npw = N // NW
    nchunks = npw // G
    NB = N // 128
    GB = G // 128
    CT = C // 8
    OROWS = CT * GB * 8
    mesh = plsc.VectorSubcoreMesh(core_axis_name="c", subcore_axis_name="s")

    @functools.partial(
        pl.kernel,
        mesh=mesh,
        compiler_params=pltpu.CompilerParams(
            use_tc_tiling_on_sc=False, needs_layout_passes=False),
        out_type=jax.ShapeDtypeStruct((CT * NB * 8, 128), jnp.float32),
        scratch_types=[
            pltpu.VMEM((2, G), jnp.float32),
            pltpu.VMEM((2, G), jnp.float32),
            pltpu.VMEM((2, G), jnp.int32),
            pltpu.VMEM((2, G), jnp.int32),
            pltpu.VMEM((2, G), jnp.int32),
            pltpu.VMEM((2, G), jnp.int32),
            pltpu.VMEM((2, G), jnp.float32),
            pltpu.VMEM((2, G), jnp.float32),
            pltpu.VMEM((2, G), jnp.float32),
            pltpu.VMEM((2, G), jnp.float32),
            pltpu.VMEM((2, G, C), jnp.float32),
            pltpu.VMEM((2, G, C), jnp.float32),
            pltpu.VMEM((2, G, C), jnp.float32),
            pltpu.VMEM((2, G, C), jnp.float32),
            pltpu.VMEM((2, OROWS, 128), jnp.float32),
            pltpu.SemaphoreType.DMA,
            pltpu.SemaphoreType.DMA,
            pltpu.SemaphoreType.DMA,
        ],
    )
    def sc_kernel(xs_hbm, ys_hbm, table_hbm, out_hbm,
                  xv, yv, i00, i01, i10, i11, w00, w01, w10, w11,
                  r00, r01, r10, r11, ov, csem, gsem, osem):
        wid = lax.axis_index("s") * info.num_cores + lax.axis_index("c")
        base0 = wid * npw
        tbl = table_hbm

        def coords_fetch(ci):
            b = ci % 2
            base = base0 + ci * G
            pltpu.async_copy(xs_hbm.at[pl.ds(base, G)], xv.at[b], csem)
            pltpu.async_copy(ys_hbm.at[pl.ds(base, G)], yv.at[b], csem)

        def coords_wait():
            pltpu.make_async_copy(xs_hbm.at[pl.ds(0, G)], xv.at[0], csem).wait()
            pltpu.make_async_copy(ys_hbm.at[pl.ds(0, G)], yv.at[0], csem).wait()

        def prep(ci):
            b = ci % 2

            @plsc.parallel_loop(0, G // L)
            def grp(j):
                s = pl.ds(j * L, L)
                x = xv[b, s]
                y = yv[b, s]
                ix = jnp.clip((x + 1.0) * (0.5 * (W - 1)), 0.0, float(W - 1))
                iy = jnp.clip((y + 1.0) * (0.5 * (H - 1)), 0.0, float(H - 1))
                x0 = ix.astype(jnp.int32)
                y0 = iy.astype(jnp.int32)
                fx = ix - x0.astype(jnp.float32)
                fy = iy - y0.astype(jnp.float32)
                x1 = jnp.minimum(x0 + 1, W - 1)
                y1 = jnp.minimum(y0 + 1, H - 1)
                b0 = y0 * W
                b1 = y1 * W
                i00[b, s] = b0 + x0
                i01[b, s] = b0 + x1
                i10[b, s] = b1 + x0
                i11[b, s] = b1 + x1
                gx = 1.0 - fx
                gy = 1.0 - fy
                w00[b, s] = gx * gy
                w01[b, s] = fx * gy
                w10[b, s] = gx * fy
                w11[b, s] = fx * fy

            pltpu.async_copy(tbl.at[i00.at[b]], r00.at[b], gsem)
            pltpu.async_copy(tbl.at[i01.at[b]], r01.at[b], gsem)
            pltpu.async_copy(tbl.at[i10.at[b]], r10.at[b], gsem)
            pltpu.async_copy(tbl.at[i11.at[b]], r11.at[b], gsem)

        def gather_wait(b):
            pltpu.make_async_copy(tbl.at[i00.at[b]], r00.at[b], gsem).wait()
            pltpu.make_async_copy(tbl.at[i01.at[b]], r01.at[b], gsem).wait()
            pltpu.make_async_copy(tbl.at[i10.at[b]], r10.at[b], gsem).wait()
            pltpu.make_async_copy(tbl.at[i11.at[b]], r11.at[b], gsem).wait()

        def blend(ci):
            b = ci % 2
            base = base0 + ci * G
            gather_wait(b)
            iota = lax.iota(jnp.int32, L)
            GB8 = GB * 8
            rowpat = (iota // 8) * GB8 + (iota % 8)

            @plsc.parallel_loop(0, G // L)
            def grp2(j):
                sw = pl.ds(j * L, L)
                a00g = w00[b, sw]
                a01g = w01[b, sw]
                a10g = w10[b, sw]
                a11g = w11[b, sw]
                tcl8 = (j // 8) * 8
                lanebase = (j % 8) * L
                rowsA = rowpat + tcl8
                rowsB = rowsA + 2 * GB8
                for k in range(L):
                    p = j * L + k
                    a00 = a00g[k]
                    a01 = a01g[k]
                    a10 = a10g[k]
                    a11 = a11g[k]
                    lane = jnp.full((L,), lanebase + k, jnp.int32)
                    for ci2, rowsv in ((0, rowsA), (1, rowsB)):
                        s = pl.ds(ci2 * L, L)
                        acc = (a00 * r00[b, p, s] + a01 * r01[b, p, s]
                               + a10 * r10[b, p, s] + a11 * r11[b, p, s])
                        plsc.store_scatter(ov.at[b], [rowsv, lane], acc)

            tcg = base // 128
            for tr in range(CT):
                pltpu.async_copy(
                    ov.at[b, pl.ds(tr * GB * 8, GB * 8), :],
                    out_hbm.at[pl.ds((tr * NB + tcg) * 8, GB * 8), :],
                    osem)

        def out_drain(b):
            for tr in range(CT):
                pltpu.make_async_copy(
                    ov.at[b, pl.ds(tr * GB * 8, GB * 8), :],
                    out_hbm.at[pl.ds(0, GB * 8), :],
                    osem).wait()

        coords_fetch(0)
        coords_fetch(1)
        coords_wait()
        prep(0)

        def body(i, carry):
            @pl.when(i + 2 < nchunks)
            def _():
                coords_fetch(i + 2)

            @pl.when(i + 1 < nchunks)
            def _():
                coords_wait()
                prep(i + 1)

            @pl.when(i >= 2)
            def _():
                out_drain(i % 2)

            blend(i)
            return carry

        lax.fori_loop(0, nchunks, body, 0)
        out_drain(nchunks % 2)
        out_drain((nchunks + 1) % 2)

    return sc_kernel


def kernel(inp, plane):
    C, H, W = plane.shape
    N = inp.shape[0]
    table = plane.transpose(1, 2, 0).reshape(H * W, C)
    xs = inp[:, 0]
    ys = inp[:, 1]
    out4 = _make_sc_bilinear(N, H * W, C, W, H)(xs, ys, table)
    NB = N // 128
    out = (out4.reshape(C // 8, NB, 8, 128)
           .transpose(0, 2, 1, 3)
           .reshape(C, N)
           .T)
    return out
